# baseline (speedup 1.0000x reference)
import functools

import jax
import jax.numpy as jnp
import numpy as np
from jax.experimental import pallas as pl
from jax.experimental.pallas import tpu as pltpu
from jax.sharding import Mesh, PartitionSpec as P

_BLK1 = 200
_BLK2 = 1000


def _stage1_body(x_ref, we_ref, wf_ref, b_ref, a_ref, adj_ref,
                 lh_ref, adjb_ref, h_scr):
    i = pl.program_id(0)

    @pl.when(i == 0)
    def _():
        xe = jax.lax.dot_general(
            x_ref[...], we_ref[...], (((1,), (0,)), ((), ())),
            preferred_element_type=jnp.float32)
        h = jax.lax.dot_general(
            xe, wf_ref[...], (((1,), (1,)), ((), ())),
            preferred_element_type=jnp.float32)
        h_scr[...] = h.astype(jnp.bfloat16)

    adjb_ref[...] = adj_ref[...].astype(jnp.bfloat16)
    t = jax.lax.dot_general(adjb_ref[...], h_scr[...], (((1,), (0,)), ((), ())),
                            preferred_element_type=jnp.float32)
    t = t + b_ref[...]
    a = a_ref[0, 0]
    lh_ref[...] = jnp.where(t >= 0.0, t, a * t).astype(jnp.bfloat16)


def _stage1(x2, adj_loc, W_emb, W_fc, bias2, a2):
    m, n = adj_loc.shape
    d = x2.shape[1]
    return pl.pallas_call(
        _stage1_body,
        grid=(m // _BLK1,),
        in_specs=[
            pl.BlockSpec((n, d), lambda i: (0, 0)),
            pl.BlockSpec((d, d), lambda i: (0, 0)),
            pl.BlockSpec((d, d), lambda i: (0, 0)),
            pl.BlockSpec((1, d), lambda i: (0, 0)),
            pl.BlockSpec((1, 1), lambda i: (0, 0)),
            pl.BlockSpec((_BLK1, n), lambda i: (i, 0)),
        ],
        out_specs=[
            pl.BlockSpec((_BLK1, d), lambda i: (i, 0)),
            pl.BlockSpec((_BLK1, n), lambda i: (i, 0)),
        ],
        out_shape=[
            jax.ShapeDtypeStruct((m, d), jnp.bfloat16),
            jax.ShapeDtypeStruct((m, n), jnp.bfloat16),
        ],
        scratch_shapes=[pltpu.VMEM((n, d), jnp.bfloat16)],
        compiler_params=pltpu.CompilerParams(
            dimension_semantics=("arbitrary",)),
    )(x2, W_emb, W_fc, bias2, a2, adj_loc)


def _prop_body(src_ref, adjb_ref, gh_ref):
    gh_ref[...] = jax.lax.dot_general(
        adjb_ref[...], src_ref[...], (((1,), (0,)), ((), ())),
        preferred_element_type=jnp.float32).astype(jnp.bfloat16)


def _final_body(src_ref, adjb_ref, lh_ref, out_ref):
    hh = lh_ref[...].astype(jnp.float32) + jax.lax.dot_general(
        adjb_ref[...], src_ref[...], (((1,), (0,)), ((), ())),
        preferred_element_type=jnp.float32)
    nrm = jnp.sqrt(jnp.sum(hh * hh, axis=-1, keepdims=True))
    out_ref[...] = hh / jnp.maximum(nrm, 1e-12)


def _prop_pass(adjb_loc, src, blk):
    m, n = adjb_loc.shape
    d = src.shape[1]
    return pl.pallas_call(
        _prop_body,
        grid=(m // blk,),
        in_specs=[
            pl.BlockSpec((n, d), lambda i: (0, 0)),
            pl.BlockSpec((blk, n), lambda i: (i, 0)),
        ],
        out_specs=pl.BlockSpec((blk, d), lambda i: (i, 0)),
        out_shape=jax.ShapeDtypeStruct((m, d), jnp.bfloat16),
        compiler_params=pltpu.CompilerParams(
            dimension_semantics=("arbitrary",)),
    )(src, adjb_loc)


def _final_pass(adjb_loc, src, lh_loc, blk):
    m, n = adjb_loc.shape
    d = src.shape[1]
    return pl.pallas_call(
        _final_body,
        grid=(m // blk,),
        in_specs=[
            pl.BlockSpec((n, d), lambda i: (0, 0)),
            pl.BlockSpec((blk, n), lambda i: (i, 0)),
            pl.BlockSpec((blk, d), lambda i: (i, 0)),
        ],
        out_specs=pl.BlockSpec((blk, d), lambda i: (i, 0)),
        out_shape=jax.ShapeDtypeStruct((m, d), jnp.float32),
        compiler_params=pltpu.CompilerParams(
            dimension_semantics=("arbitrary",)),
    )(src, adjb_loc, lh_loc)


def _stage2_body(blk2, lh_ref, adjb_ref, out_ref, g0, g1, gb):
    s = pl.program_id(0)
    i = pl.program_id(1)
    row = i * blk2

    @pl.when(((s == 1) | (s == 3)) & (i == 0))
    def _():
        gb[...] = g0[...].astype(jnp.bfloat16)

    @pl.when(((s == 2) | (s == 4)) & (i == 0))
    def _():
        gb[...] = g1[...].astype(jnp.bfloat16)

    def prop(src):
        return jax.lax.dot_general(adjb_ref[...], src, (((1,), (0,)), ((), ())),
                                   preferred_element_type=jnp.float32)

    @pl.when(s == 0)
    def _():
        g0[pl.ds(row, blk2), :] = prop(lh_ref[...])

    @pl.when((s == 1) | (s == 3))
    def _():
        g1[pl.ds(row, blk2), :] = prop(gb[...])

    @pl.when(s == 2)
    def _():
        g0[pl.ds(row, blk2), :] = prop(gb[...])

    @pl.when(s == 4)
    def _():
        hh = lh_ref[pl.ds(row, blk2), :].astype(jnp.float32) + prop(gb[...])
        nrm = jnp.sqrt(jnp.sum(hh * hh, axis=-1, keepdims=True))
        out_ref[...] = hh / jnp.maximum(nrm, 1e-12)


def _single_device(x2, adj2, W_emb, W_fc, bias2, a2):
    n, d = x2.shape
    lh, adjb = _stage1(x2, adj2, W_emb, W_fc, bias2, a2)
    blk2 = min(_BLK2, n)
    return pl.pallas_call(
        functools.partial(_stage2_body, blk2),
        grid=(5, n // blk2),
        in_specs=[
            pl.BlockSpec((n, d), lambda s, i: (0, 0)),
            pl.BlockSpec((blk2, n), lambda s, i: (i, 0)),
        ],
        out_specs=pl.BlockSpec(
            (blk2, d), lambda s, i: (jnp.where(s == 4, i, 0), 0)),
        out_shape=jax.ShapeDtypeStruct((n, d), jnp.float32),
        scratch_shapes=[
            pltpu.VMEM((n, d), jnp.float32),
            pltpu.VMEM((n, d), jnp.float32),
            pltpu.VMEM((n, d), jnp.bfloat16),
        ],
        compiler_params=pltpu.CompilerParams(
            dimension_semantics=("arbitrary", "arbitrary")),
    )(lh, adjb)


def _sharded(x2, adj2, W_emb, W_fc, bias2, a2, devs):
    mesh = Mesh(np.array(devs[:2]), ("x",))

    def shard_fn(x_full, adj_loc, we, wf, b2, a2):
        lh_loc, adjb_loc = _stage1(x_full, adj_loc, we, wf, b2, a2)
        g = jax.lax.all_gather(lh_loc, "x", axis=0, tiled=True)
        for _ in range(4):
            gh = _prop_pass(adjb_loc, g, _BLK2)
            g = jax.lax.all_gather(gh, "x", axis=0, tiled=True)
        return _final_pass(adjb_loc, g, lh_loc, _BLK2)

    return jax.shard_map(
        shard_fn, mesh=mesh,
        in_specs=(P(), P("x", None), P(), P(), P(), P()),
        out_specs=P("x", None), check_vma=False,
    )(x2, adj2, W_emb, W_fc, bias2, a2)


def kernel(x, adj, W_emb, W_fc, bias, prelu_a):
    n = x.shape[1]
    d = x.shape[2]
    x2 = x.reshape(n, d)
    adj2 = adj.reshape(n, n)
    bias2 = bias.reshape(1, d)
    a2 = jnp.reshape(prelu_a, (1, 1)).astype(jnp.float32)

    devs = jax.devices()
    half = n // 2
    can_shard = (len(devs) >= 2 and n % 2 == 0
                 and half % _BLK2 == 0 and half % _BLK1 == 0)
    if can_shard:
        out = _sharded(x2, adj2, W_emb, W_fc, bias2, a2, devs)
    else:
        out = _single_device(x2, adj2, W_emb, W_fc, bias2, a2)
    return out[None, :, :]


# speedup vs baseline: 1.9416x; 1.9416x over previous
import functools

import jax
import jax.numpy as jnp
from jax.experimental import pallas as pl
from jax.experimental.pallas import tpu as pltpu

_BLK1 = 400
_BLK2 = 1000


def _stage1_body(x_ref, we_ref, wf_ref, b_ref, a_ref, adj_ref,
                 lh_ref, adjb_ref, h_scr):
    i = pl.program_id(0)

    @pl.when(i == 0)
    def _():
        xe = jax.lax.dot_general(
            x_ref[...], we_ref[...], (((1,), (0,)), ((), ())),
            preferred_element_type=jnp.float32)
        h = jax.lax.dot_general(
            xe, wf_ref[...], (((1,), (1,)), ((), ())),
            preferred_element_type=jnp.float32)
        h_scr[...] = h.astype(jnp.bfloat16)

    adjb_ref[...] = adj_ref[...].astype(jnp.bfloat16)
    t = jax.lax.dot_general(adjb_ref[...], h_scr[...], (((1,), (0,)), ((), ())),
                            preferred_element_type=jnp.float32)
    t = t + b_ref[...]
    a = a_ref[0, 0]
    lh_ref[...] = jnp.where(t >= 0.0, t, a * t).astype(jnp.bfloat16)


def _stage1(x2, adj_loc, W_emb, W_fc, bias2, a2):
    m, n = adj_loc.shape
    d = x2.shape[1]
    return pl.pallas_call(
        _stage1_body,
        grid=(m // _BLK1,),
        in_specs=[
            pl.BlockSpec((n, d), lambda i: (0, 0)),
            pl.BlockSpec((d, d), lambda i: (0, 0)),
            pl.BlockSpec((d, d), lambda i: (0, 0)),
            pl.BlockSpec((1, d), lambda i: (0, 0)),
            pl.BlockSpec((1, 1), lambda i: (0, 0)),
            pl.BlockSpec((_BLK1, n), lambda i: (i, 0)),
        ],
        out_specs=[
            pl.BlockSpec((_BLK1, d), lambda i: (i, 0)),
            pl.BlockSpec((_BLK1, n), lambda i: (i, 0)),
        ],
        out_shape=[
            jax.ShapeDtypeStruct((m, d), jnp.bfloat16),
            jax.ShapeDtypeStruct((m, n), jnp.bfloat16),
        ],
        scratch_shapes=[pltpu.VMEM((n, d), jnp.bfloat16)],
        compiler_params=pltpu.CompilerParams(
            dimension_semantics=("arbitrary",)),
    )(x2, W_emb, W_fc, bias2, a2, adj_loc)


def _stage2_body(blk2, lh_ref, adjb_ref, out_ref, g0, g1, gb):
    s = pl.program_id(0)
    i = pl.program_id(1)
    row = i * blk2

    @pl.when(((s == 1) | (s == 3)) & (i == 0))
    def _():
        gb[...] = g0[...].astype(jnp.bfloat16)

    @pl.when(((s == 2) | (s == 4)) & (i == 0))
    def _():
        gb[...] = g1[...].astype(jnp.bfloat16)

    def prop(src):
        return jax.lax.dot_general(adjb_ref[...], src, (((1,), (0,)), ((), ())),
                                   preferred_element_type=jnp.float32)

    @pl.when(s == 0)
    def _():
        g0[pl.ds(row, blk2), :] = prop(lh_ref[...])

    @pl.when((s == 1) | (s == 3))
    def _():
        g1[pl.ds(row, blk2), :] = prop(gb[...])

    @pl.when(s == 2)
    def _():
        g0[pl.ds(row, blk2), :] = prop(gb[...])

    @pl.when(s == 4)
    def _():
        hh = lh_ref[pl.ds(row, blk2), :].astype(jnp.float32) + prop(gb[...])
        nrm = jnp.sqrt(jnp.sum(hh * hh, axis=-1, keepdims=True))
        out_ref[...] = hh / jnp.maximum(nrm, 1e-12)


def _single_device(x2, adj2, W_emb, W_fc, bias2, a2):
    n, d = x2.shape
    lh, adjb = _stage1(x2, adj2, W_emb, W_fc, bias2, a2)
    blk2 = min(_BLK2, n)
    return pl.pallas_call(
        functools.partial(_stage2_body, blk2),
        grid=(5, n // blk2),
        in_specs=[
            pl.BlockSpec((n, d), lambda s, i: (0, 0)),
            pl.BlockSpec((blk2, n), lambda s, i: (i, 0)),
        ],
        out_specs=pl.BlockSpec(
            (blk2, d), lambda s, i: (jnp.where(s == 4, i, 0), 0)),
        out_shape=jax.ShapeDtypeStruct((n, d), jnp.float32),
        scratch_shapes=[
            pltpu.VMEM((n, d), jnp.float32),
            pltpu.VMEM((n, d), jnp.float32),
            pltpu.VMEM((n, d), jnp.bfloat16),
        ],
        compiler_params=pltpu.CompilerParams(
            dimension_semantics=("arbitrary", "arbitrary")),
    )(lh, adjb)


def kernel(x, adj, W_emb, W_fc, bias, prelu_a):
    n = x.shape[1]
    d = x.shape[2]
    x2 = x.reshape(n, d)
    adj2 = adj.reshape(n, n)
    bias2 = bias.reshape(1, d)
    a2 = jnp.reshape(prelu_a, (1, 1)).astype(jnp.float32)

    out = _single_device(x2, adj2, W_emb, W_fc, bias2, a2)
    return out[None, :, :]
